# baseline (speedup 1.0000x reference)
import numpy as np

import jax
import jax.numpy as jnp
from jax.experimental import pallas as pl
from jax.experimental.pallas import tpu as pltpu

_NUM_FEATURES = 17
_EPS = 1e-10
_SCALES = np.linspace(0.5, 1.5, 15).astype(np.float32)
_OFFSETS = np.linspace(0.1, 1.5, 15).astype(np.float32)

_TWO_PI = 2.0 * np.pi
_CW1 = float(np.float32(np.trunc(_TWO_PI * 512) / 512))
_CW2 = float(np.float32(np.trunc((_TWO_PI - _CW1) * 2**20) / 2**20))
_CW3 = float(np.float32(_TWO_PI - _CW1 - _CW2))
_INV_2PI = float(np.float32(1.0 / _TWO_PI))
_COS_POLY = [1.0, -0.5, 0.0416666679084301, -0.00138888880610466,
             2.480154398654122e-05, -2.7556220061342174e-07,
             2.086000039369651e-09, -1.1321093368321655e-11,
             4.0492382938437516e-14]


def _fast_cos(t):
    k = jnp.round(t * _INV_2PI)
    r = ((t - k * _CW1) - k * _CW2) - k * _CW3
    x = r * r
    acc = jnp.full_like(x, _COS_POLY[8])
    for c in _COS_POLY[7::-1]:
        acc = acc * x + c
    return acc


def _expansion_mats():
    glo = np.zeros((128, 128), np.float32)
    ghi = np.zeros((128, 128), np.float32)
    for t in range(64):
        glo[t, 2 * t] = 1.0
        glo[t, 2 * t + 1] = 1.0
    for t in range(64, 128):
        ghi[t, 2 * t - 128] = 1.0
        ghi[t, 2 * t - 127] = 1.0
    manc = np.zeros((128, 128), np.float32)
    for m in range(1, 128):
        mp = m
        while mp >= 1:
            manc[mp, m] = 1.0
            mp >>= 1
    return glo, ghi, manc


_GLO_NP, _GHI_NP, _MANC_NP = _expansion_mats()


def _dot(a, b):
    return jax.lax.dot(a, b, precision=jax.lax.Precision.HIGHEST)


def _dot3(a, b):
    return jax.lax.dot_general(
        a, b, (((2,), (0,)), ((), ())),
        precision=jax.lax.Precision.HIGHEST)


def _expand(prev, glo, ghi, nimg):
    rp = prev.shape[1]
    lo = _dot3(prev, glo)
    hi = _dot3(prev, ghi)
    return jnp.stack([lo, hi], axis=2).reshape(nimg, 2 * rp, 128)


def _body(coef_ref, x_ref, glo_ref, ghi_ref, manc_ref, out_ref):
    x = x_ref[...]
    glo = glo_ref[...]
    ghi = ghi_ref[...]
    manc = manc_ref[...]
    nimg, nrow, _ = x.shape

    lane = jax.lax.broadcasted_iota(jnp.int32, (nimg, nrow, 128), 2)
    row = jax.lax.broadcasted_iota(jnp.int32, (nimg, nrow, 128), 1)

    lane_r = jnp.roll(x, 1, axis=2)
    row_r = jnp.roll(lane_r, 1, axis=1)
    vmain = jnp.where(lane == 0, row_r, lane_r)
    vmain = jnp.where((lane == 0) & (row == 0), 0.0, vmain)
    vtail = jnp.roll(x[:, nrow - 1:nrow, :], 1, axis=2)

    s = [float(x) for x in _SCALES]
    o = [float(x) for x in _OFFSETS]

    def coef(j):
        return coef_ref[:, :, j:j + 1]

    def score(t):
        a = jnp.abs(t)
        lin = coef(0) * a + coef(1)
        for j in range(9):
            k = 6 + j
            lin = lin + coef(2 + j) * jnp.log(a * s[k] + (o[k] + _EPS))
        t7 = a * s[7] + o[7]
        lshape = t7 * jax.lax.rsqrt(t7 * (a * s[6] + o[6]))
        lin = lin + coef(11) * lshape
        lin = lin + coef(12) * _fast_cos(a * s[5] + coef(13))
        return 0.5 + 0.5 * jnp.tanh(0.5 * lin)

    vpar = _expand(vmain[:, 0:256], glo, ghi, nimg)
    c1 = (vmain - vpar) * score(vmain)
    vpar_t = _dot(vmain[:, 256, :], glo).reshape(nimg, 1, 128)
    c1t = (vtail - vpar_t) * score(vtail)

    out0 = _dot(c1[:, 0, :], manc)
    pieces = [out0.reshape(nimg, 1, 128)]
    prev = (_dot(out0, ghi) + c1[:, 1, :]).reshape(nimg, 1, 128)
    pieces.append(prev)
    rs = 2
    for d in range(8, 16):
        rp = 1 << (d - 8)
        prev = _expand(prev, glo, ghi, nimg) + c1[:, rs:rs + 2 * rp]
        pieces.append(prev)
        rs += 2 * rp
    pieces.append(_dot(prev[:, 0, :], glo).reshape(nimg, 1, 128) + c1t)

    full = jnp.concatenate(pieces, axis=1)
    lane_l = jnp.roll(full, -1, axis=2)
    row_l = jnp.roll(lane_l, -1, axis=1)
    lane2 = jax.lax.broadcasted_iota(jnp.int32, lane_l.shape, 2)
    shifted = jnp.where(lane2 == 127, row_l, lane_l)
    out_ref[...] = shifted[:, 0:nrow]


def kernel(batched_input, weight, bias):
    B, N, H, W = batched_input.shape
    n = H * W
    nimg = B * N

    w = weight[..., 0]
    sc = jnp.asarray(_SCALES[:5])
    of = jnp.asarray(_OFFSETS[:5])
    cA = (w[:, :5] * sc[None, :]).sum(axis=1)
    cB = (w[:, :5] * of[None, :]).sum(axis=1) + bias[:, 0]
    cR = jnp.sqrt(w[:, 15] ** 2 + w[:, 16] ** 2)
    cPhi = float(_OFFSETS[5]) - jnp.arctan2(w[:, 16], w[:, 15])
    coefs = jnp.concatenate(
        [cA[:, None], cB[:, None], w[:, 5:14], w[:, 14:15],
         cR[:, None], cPhi[:, None],
         jnp.zeros((N, 114), jnp.float32)], axis=1)
    coefs = jnp.tile(coefs, (B, 1)).reshape(nimg, 1, 128)

    x = batched_input.reshape(nimg, n // 128, 128)

    glo = jnp.asarray(_GLO_NP)
    ghi = jnp.asarray(_GHI_NP)
    manc = jnp.asarray(_MANC_NP)

    ngrid = 4
    bi = nimg // ngrid
    out = pl.pallas_call(
        _body,
        grid=(ngrid,),
        in_specs=[
            pl.BlockSpec((bi, 1, 128), lambda g: (g, 0, 0)),
            pl.BlockSpec((bi, n // 128, 128), lambda g: (g, 0, 0)),
            pl.BlockSpec((128, 128), lambda g: (0, 0)),
            pl.BlockSpec((128, 128), lambda g: (0, 0)),
            pl.BlockSpec((128, 128), lambda g: (0, 0)),
        ],
        out_specs=pl.BlockSpec((bi, n // 128, 128), lambda g: (g, 0, 0)),
        out_shape=jax.ShapeDtypeStruct((nimg, n // 128, 128), jnp.float32),
        compiler_params=pltpu.CompilerParams(
            dimension_semantics=("parallel",)),
    )(coefs, x, glo, ghi, manc)

    return out.reshape(B, N, H, W)


# speedup vs baseline: 1.0656x; 1.0656x over previous
import numpy as np

import jax
import jax.numpy as jnp
from jax.experimental import pallas as pl
from jax.experimental.pallas import tpu as pltpu

_NUM_FEATURES = 17
_EPS = 1e-10
_SCALES = np.linspace(0.5, 1.5, 15).astype(np.float32)
_OFFSETS = np.linspace(0.1, 1.5, 15).astype(np.float32)

_TWO_PI = 2.0 * np.pi
_CW1 = float(np.float32(np.trunc(_TWO_PI * 512) / 512))
_CW2 = float(np.float32(np.trunc((_TWO_PI - _CW1) * 2**20) / 2**20))
_CW3 = float(np.float32(_TWO_PI - _CW1 - _CW2))
_INV_2PI = float(np.float32(1.0 / _TWO_PI))
_COS_POLY = [1.0, -0.5, 0.0416666679084301, -0.00138888880610466,
             2.480154398654122e-05, -2.7556220061342174e-07,
             2.086000039369651e-09, -1.1321093368321655e-11,
             4.0492382938437516e-14]


_RK = (_OFFSETS[6:15].astype(np.float64) / _SCALES[6:15].astype(np.float64))
_RBAR = float((_RK.min() + _RK.max()) / 2)
_LOG_J = 5
_DM = np.stack([((-1.0) ** (j + 1) / j) * (_RK - _RBAR) ** j
                for j in range(1, _LOG_J + 1)], axis=1).astype(np.float32)
_LNS = np.log(_SCALES[6:15].astype(np.float64)).astype(np.float32)


def _fast_cos(t):
    k = jnp.round(t * _INV_2PI)
    r = ((t - k * _CW1) - k * _CW2) - k * _CW3
    x = r * r
    acc = jnp.full_like(x, _COS_POLY[8])
    for c in _COS_POLY[7::-1]:
        acc = acc * x + c
    return acc


def _expansion_mats():
    glo = np.zeros((128, 128), np.float32)
    ghi = np.zeros((128, 128), np.float32)
    for t in range(64):
        glo[t, 2 * t] = 1.0
        glo[t, 2 * t + 1] = 1.0
    for t in range(64, 128):
        ghi[t, 2 * t - 128] = 1.0
        ghi[t, 2 * t - 127] = 1.0
    manc = np.zeros((128, 128), np.float32)
    for m in range(1, 128):
        mp = m
        while mp >= 1:
            manc[mp, m] = 1.0
            mp >>= 1
    return glo, ghi, manc


_GLO_NP, _GHI_NP, _MANC_NP = _expansion_mats()


def _dot(a, b):
    return jax.lax.dot(a, b, precision=jax.lax.Precision.HIGHEST)


def _dot3(a, b):
    return jax.lax.dot_general(
        a, b, (((2,), (0,)), ((), ())),
        precision=jax.lax.Precision.HIGHEST)


def _expand(prev, glo, ghi, nimg):
    rp = prev.shape[1]
    lo = _dot3(prev, glo)
    hi = _dot3(prev, ghi)
    return jnp.stack([lo, hi], axis=2).reshape(nimg, 2 * rp, 128)


def _body(coef_ref, x_ref, glo_ref, ghi_ref, manc_ref, out_ref):
    x = x_ref[...]
    glo = glo_ref[...]
    ghi = ghi_ref[...]
    manc = manc_ref[...]
    nimg, nrow, _ = x.shape

    lane = jax.lax.broadcasted_iota(jnp.int32, (nimg, nrow, 128), 2)
    row = jax.lax.broadcasted_iota(jnp.int32, (nimg, nrow, 128), 1)

    lane_r = jnp.roll(x, 1, axis=2)
    row_r = jnp.roll(lane_r, 1, axis=1)
    vmain = jnp.where(lane == 0, row_r, lane_r)
    vmain = jnp.where((lane == 0) & (row == 0), 0.0, vmain)
    vtail = jnp.roll(x[:, nrow - 1:nrow, :], 1, axis=2)

    s = [float(x) for x in _SCALES]
    o = [float(x) for x in _OFFSETS]

    def coef(j):
        return coef_ref[:, :, j:j + 1]

    def score(t):
        a = jnp.abs(t)
        t0 = a + _RBAR
        lin = coef(0) * a + coef(1) + coef(2) * jnp.log(t0)
        iv = jax.lax.rsqrt(t0)
        iv = iv * iv
        h = coef(7)
        for j in range(6, 2, -1):
            h = h * iv + coef(j)
        lin = lin + h * iv
        t7 = a * s[7] + o[7]
        lshape = t7 * jax.lax.rsqrt(t7 * (a * s[6] + o[6]))
        lin = lin + coef(8) * lshape
        lin = lin + coef(9) * _fast_cos(a * s[5] + coef(10))
        return 0.5 + 0.5 * jnp.tanh(0.5 * lin)

    vpar = _expand(vmain[:, 0:256], glo, ghi, nimg)
    c1 = (vmain - vpar) * score(vmain)
    vpar_t = _dot(vmain[:, 256, :], glo).reshape(nimg, 1, 128)
    c1t = (vtail - vpar_t) * score(vtail)

    out0 = _dot(c1[:, 0, :], manc)
    pieces = [out0.reshape(nimg, 1, 128)]
    prev = (_dot(out0, ghi) + c1[:, 1, :]).reshape(nimg, 1, 128)
    pieces.append(prev)
    rs = 2
    for d in range(8, 16):
        rp = 1 << (d - 8)
        prev = _expand(prev, glo, ghi, nimg) + c1[:, rs:rs + 2 * rp]
        pieces.append(prev)
        rs += 2 * rp
    pieces.append(_dot(prev[:, 0, :], glo).reshape(nimg, 1, 128) + c1t)

    full = jnp.concatenate(pieces, axis=1)
    lane_l = jnp.roll(full, -1, axis=2)
    row_l = jnp.roll(lane_l, -1, axis=1)
    lane2 = jax.lax.broadcasted_iota(jnp.int32, lane_l.shape, 2)
    shifted = jnp.where(lane2 == 127, row_l, lane_l)
    out_ref[...] = shifted[:, 0:nrow]


def kernel(batched_input, weight, bias):
    B, N, H, W = batched_input.shape
    n = H * W
    nimg = B * N

    w = weight[..., 0]
    sc = jnp.asarray(_SCALES[:5])
    of = jnp.asarray(_OFFSETS[:5])
    cA = (w[:, :5] * sc[None, :]).sum(axis=1)
    cB = (w[:, :5] * of[None, :]).sum(axis=1) + bias[:, 0]
    wl = w[:, 5:14]
    cB = cB + (wl * jnp.asarray(_LNS)[None, :]).sum(axis=1)
    cL = wl.sum(axis=1)
    m = wl @ jnp.asarray(_DM)
    cR = jnp.sqrt(w[:, 15] ** 2 + w[:, 16] ** 2)
    cPhi = float(_OFFSETS[5]) - jnp.arctan2(w[:, 16], w[:, 15])
    coefs = jnp.concatenate(
        [cA[:, None], cB[:, None], cL[:, None], m, w[:, 14:15],
         cR[:, None], cPhi[:, None],
         jnp.zeros((N, 117), jnp.float32)], axis=1)
    coefs = jnp.tile(coefs, (B, 1)).reshape(nimg, 1, 128)

    x = batched_input.reshape(nimg, n // 128, 128)

    glo = jnp.asarray(_GLO_NP)
    ghi = jnp.asarray(_GHI_NP)
    manc = jnp.asarray(_MANC_NP)

    out = pl.pallas_call(
        _body,
        in_specs=[
            pl.BlockSpec((nimg, 1, 128), lambda: (0, 0, 0)),
            pl.BlockSpec((nimg, n // 128, 128), lambda: (0, 0, 0)),
            pl.BlockSpec((128, 128), lambda: (0, 0)),
            pl.BlockSpec((128, 128), lambda: (0, 0)),
            pl.BlockSpec((128, 128), lambda: (0, 0)),
        ],
        out_specs=pl.BlockSpec((nimg, n // 128, 128), lambda: (0, 0, 0)),
        out_shape=jax.ShapeDtypeStruct((nimg, n // 128, 128), jnp.float32),
    )(coefs, x, glo, ghi, manc)

    return out.reshape(B, N, H, W)


# speedup vs baseline: 1.1485x; 1.0778x over previous
import numpy as np

import jax
import jax.numpy as jnp
from jax.experimental import pallas as pl
from jax.experimental.pallas import tpu as pltpu

_NUM_FEATURES = 17
_EPS = 1e-10
_SCALES = np.linspace(0.5, 1.5, 15).astype(np.float32)
_OFFSETS = np.linspace(0.1, 1.5, 15).astype(np.float32)

_TWO_PI = 2.0 * np.pi
_CW1 = float(np.float32(np.trunc(_TWO_PI * 512) / 512))
_CW2 = float(np.float32(np.trunc((_TWO_PI - _CW1) * 2**20) / 2**20))
_CW3 = float(np.float32(_TWO_PI - _CW1 - _CW2))
_INV_2PI = float(np.float32(1.0 / _TWO_PI))
_COS_POLY = [1.0, -0.5, 0.0416666679084301, -0.00138888880610466,
             2.480154398654122e-05, -2.7556220061342174e-07,
             2.086000039369651e-09, -1.1321093368321655e-11,
             4.0492382938437516e-14]


_RK = (_OFFSETS[6:15].astype(np.float64) / _SCALES[6:15].astype(np.float64))
_RBAR = float((_RK.min() + _RK.max()) / 2)
_LOG_J = 5
_DM = np.stack([((-1.0) ** (j + 1) / j) * (_RK - _RBAR) ** j
                for j in range(1, _LOG_J + 1)], axis=1).astype(np.float32)
_LNS = np.log(_SCALES[6:15].astype(np.float64)).astype(np.float32)


def _fast_cos(t):
    k = jnp.round(t * _INV_2PI)
    r = ((t - k * _CW1) - k * _CW2) - k * _CW3
    x = r * r
    acc = jnp.full_like(x, _COS_POLY[8])
    for c in _COS_POLY[7::-1]:
        acc = acc * x + c
    return acc


def _expansion_mats():
    glo = np.zeros((128, 128), np.float32)
    ghi = np.zeros((128, 128), np.float32)
    for t in range(64):
        glo[t, 2 * t] = 1.0
        glo[t, 2 * t + 1] = 1.0
    for t in range(64, 128):
        ghi[t, 2 * t - 128] = 1.0
        ghi[t, 2 * t - 127] = 1.0
    manc = np.zeros((128, 128), np.float32)
    for m in range(1, 128):
        mp = m
        while mp >= 1:
            manc[mp, m] = 1.0
            mp >>= 1
    return glo, ghi, manc


_GLO_NP, _GHI_NP, _MANC_NP = _expansion_mats()


def _dot(a, b):
    return jax.lax.dot(a, b, precision=jax.lax.Precision.HIGHEST)


def _dot3(a, b):
    return jax.lax.dot_general(
        a, b, (((2,), (0,)), ((), ())),
        precision=jax.lax.Precision.HIGHEST)


def _expand(prev, glo, ghi, nimg):
    rp = prev.shape[1]
    lo = _dot3(prev, glo)
    hi = _dot3(prev, ghi)
    return jnp.stack([lo, hi], axis=2).reshape(nimg, 2 * rp, 128)


def _body(coef_ref, x_ref, glo_ref, ghi_ref, manc_ref, out_ref):
    x = x_ref[...]
    glo = glo_ref[...]
    ghi = ghi_ref[...]
    manc = manc_ref[...]
    nimg, nrow, _ = x.shape

    lane = jax.lax.broadcasted_iota(jnp.int32, (nimg, nrow, 128), 2)
    row = jax.lax.broadcasted_iota(jnp.int32, (nimg, nrow, 128), 1)

    lane_r = jnp.roll(x, 1, axis=2)
    row_r = jnp.roll(lane_r, 1, axis=1)
    vmain = jnp.where(lane == 0, row_r, lane_r)
    vmain = jnp.where((lane == 0) & (row == 0), 0.0, vmain)
    vtail = jnp.roll(x[:, nrow - 1:nrow, :], 1, axis=2)

    s = [float(x) for x in _SCALES]
    o = [float(x) for x in _OFFSETS]

    def coef(j):
        return coef_ref[:, :, j:j + 1]

    def score(t):
        a = jnp.abs(t)
        t0 = a + _RBAR
        lin = coef(0) * a + coef(1) + coef(2) * jnp.log(t0)
        iv = jax.lax.rsqrt(t0)
        iv = iv * iv
        h = coef(7)
        for j in range(6, 2, -1):
            h = h * iv + coef(j)
        lin = lin + h * iv
        t7 = a * s[7] + o[7]
        lshape = t7 * jax.lax.rsqrt(t7 * (a * s[6] + o[6]))
        lin = lin + coef(8) * lshape
        lin = lin + coef(9) * _fast_cos(a * s[5] + coef(10))
        return 0.5 + 0.5 * jnp.tanh(0.5 * lin)

    vpar = _expand(vmain[:, 0:256], glo, ghi, nimg)
    c1 = (vmain - vpar) * score(vmain)
    vpar_t = _dot(vmain[:, 256, :], glo).reshape(nimg, 1, 128)
    c1t = (vtail - vpar_t) * score(vtail)

    out0 = _dot(c1[:, 0, :], manc)
    pieces = [out0.reshape(nimg, 1, 128)]
    prev = (_dot(out0, ghi) + c1[:, 1, :]).reshape(nimg, 1, 128)
    pieces.append(prev)
    rs = 2
    for d in range(8, 16):
        rp = 1 << (d - 8)
        prev = _expand(prev, glo, ghi, nimg) + c1[:, rs:rs + 2 * rp]
        pieces.append(prev)
        rs += 2 * rp
    pieces.append(_dot(prev[:, 0, :], glo).reshape(nimg, 1, 128) + c1t)

    full = jnp.concatenate(pieces, axis=1)
    lane_l = jnp.roll(full, -1, axis=2)
    row_l = jnp.roll(lane_l, -1, axis=1)
    lane2 = jax.lax.broadcasted_iota(jnp.int32, lane_l.shape, 2)
    shifted = jnp.where(lane2 == 127, row_l, lane_l)
    out_ref[...] = shifted[:, 0:nrow]


def kernel(batched_input, weight, bias):
    B, N, H, W = batched_input.shape
    n = H * W
    nimg = B * N

    w = weight[..., 0]
    M = np.zeros((18, 128), np.float32)
    M[0:5, 0] = _SCALES[:5]
    M[0:5, 1] = _OFFSETS[:5]
    M[17, 1] = 1.0
    M[5:14, 1] = _LNS
    M[5:14, 2] = 1.0
    M[5:14, 3:8] = _DM
    M[14, 8] = 1.0
    e9 = np.zeros((128,), np.float32)
    e10 = np.zeros((128,), np.float32)
    e9[9] = 1.0
    e10[10] = 1.0
    lin3 = jnp.concatenate([w, bias], axis=1) @ jnp.asarray(M)
    cR = jnp.sqrt(w[:, 15] ** 2 + w[:, 16] ** 2)
    cPhi = float(_OFFSETS[5]) - jnp.arctan2(w[:, 16], w[:, 15])
    coefs = (lin3 + cR[:, None] * jnp.asarray(e9)[None, :]
             + cPhi[:, None] * jnp.asarray(e10)[None, :])
    coefs = jnp.tile(coefs, (B, 1)).reshape(nimg, 1, 128)

    x = batched_input.reshape(nimg, n // 128, 128)

    glo = jnp.asarray(_GLO_NP)
    ghi = jnp.asarray(_GHI_NP)
    manc = jnp.asarray(_MANC_NP)

    out = pl.pallas_call(
        _body,
        in_specs=[
            pl.BlockSpec((nimg, 1, 128), lambda: (0, 0, 0)),
            pl.BlockSpec((nimg, n // 128, 128), lambda: (0, 0, 0)),
            pl.BlockSpec((128, 128), lambda: (0, 0)),
            pl.BlockSpec((128, 128), lambda: (0, 0)),
            pl.BlockSpec((128, 128), lambda: (0, 0)),
        ],
        out_specs=pl.BlockSpec((nimg, n // 128, 128), lambda: (0, 0, 0)),
        out_shape=jax.ShapeDtypeStruct((nimg, n // 128, 128), jnp.float32),
    )(coefs, x, glo, ghi, manc)

    return out.reshape(B, N, H, W)
